# baseline (speedup 1.0000x reference)
import jax
import jax.numpy as jnp
from jax import lax
from jax.experimental import pallas as pl
from jax.experimental.pallas import tpu as pltpu
from jax.experimental.pallas import tpu_sc as plsc

N = 10000
H = 128
E = 320000

NC = 2
NS = 16
NW = NC * NS
CH = 80
NCHUNK = 126
GRP = 3
NG = NCHUNK // GRP
EPAD = NW * NCHUNK * CH - E
CHC = 112
NCHUNK_C = NCHUNK * CH // CHC
NP = 10112
WPT = NP // NS

_MESH = plsc.VectorSubcoreMesh(
    core_axis_name="c", subcore_axis_name="s", num_cores=NC, num_subcores=NS
)


def _zero16():
    return jnp.zeros((16,), jnp.float32)


def _sc_count_body(dst_hbm, out_hbm, idx_v, ones_v, zero_v, tmp_v, cnt_sh):
    c = lax.axis_index("c")
    s = lax.axis_index("s")
    wid = s * NC + c
    for j in range(CHC // 16):
        ones_v[pl.ds(j * 16, 16)] = jnp.full((16,), 1.0, jnp.float32)
    for j in range(8):
        zero_v[pl.ds(j * 16, 16)] = _zero16()
    base = s * WPT
    for k in range(4):
        pltpu.sync_copy(zero_v, cnt_sh.at[pl.ds(base + k * 128, 128)])
    pltpu.sync_copy(zero_v.at[pl.ds(0, 120)], cnt_sh.at[pl.ds(base + 512, 120)])
    plsc.subcore_barrier()
    pltpu.sync_copy(dst_hbm.at[wid], idx_v)

    def body(i, carry):
        pltpu.sync_copy(ones_v, cnt_sh.at[idx_v.at[i]], add=True)
        return carry

    lax.fori_loop(0, NCHUNK_C, body, 0)
    plsc.subcore_barrier()
    pltpu.sync_copy(cnt_sh.at[pl.ds(base, WPT)], tmp_v)
    pltpu.sync_copy(tmp_v, out_hbm.at[pl.ds(c * NP + base, WPT)])


_sc_count = pl.kernel(
    _sc_count_body,
    out_type=jax.ShapeDtypeStruct((NC * NP,), jnp.float32),
    mesh=_MESH,
    scratch_types=[
        pltpu.VMEM((NCHUNK_C, CHC), jnp.int32),
        pltpu.VMEM((CHC,), jnp.float32),
        pltpu.VMEM((128,), jnp.float32),
        pltpu.VMEM((WPT,), jnp.float32),
        pltpu.VMEM_SHARED((NP,), jnp.float32),
    ],
)


def _sc_scatter_body(y_hbm, pk_hbm, out_hbm, pk_v, sidx_v, didx_v, buf_v,
                     agg_sh, g0, g1, g2):
    gsems = (g0, g1, g2)
    c = lax.axis_index("c")
    s = lax.axis_index("s")
    wid = s * NC + c

    def zbody(r, carry):
        for j in range(8):
            buf_v[0, r, pl.ds(j * 16, 16)] = _zero16()
        return carry

    lax.fori_loop(0, CH, zbody, 0)
    base = s * WPT
    for k in range(7):
        pltpu.sync_copy(buf_v.at[0], agg_sh.at[pl.ds(base + k * CH, CH)])
    pltpu.sync_copy(buf_v.at[0, pl.ds(0, 72)],
                    agg_sh.at[pl.ds(base + 7 * CH, 72)])
    plsc.subcore_barrier()
    pltpu.sync_copy(pk_hbm.at[wid], pk_v)

    def unpack(i, b):
        for j in range(CH // 16):
            v = pk_v[i, pl.ds(j * 16, 16)]
            sidx_v[b, pl.ds(j * 16, 16)] = jnp.bitwise_and(
                v, jnp.int32(0xFFFF))
            didx_v[b, pl.ds(j * 16, 16)] = lax.shift_right_logical(v, 16)

    def fire_g(b):
        pltpu.async_copy(y_hbm.at[sidx_v.at[b]], buf_v.at[b], gsems[b])

    def wait_g(b):
        pltpu.make_async_copy(y_hbm.at[sidx_v.at[b]], buf_v.at[b],
                              gsems[b]).wait()

    for b in range(GRP):
        unpack(b, b)
        fire_g(b)

    def body(g, carry):
        i0 = g * GRP
        for b in range(GRP):
            wait_g(b)
            pltpu.sync_copy(buf_v.at[b], agg_sh.at[didx_v.at[b]], add=True)

            @pl.when(g < NG - 1)
            def _(b=b):
                unpack(i0 + GRP + b, b)
                fire_g(b)

        return carry

    lax.fori_loop(0, NG, body, 0)
    plsc.subcore_barrier()
    pltpu.sync_copy(agg_sh.at[pl.ds(base, WPT)], out_hbm.at[c, pl.ds(base, WPT)])


_sc_scatter = pl.kernel(
    _sc_scatter_body,
    out_type=jax.ShapeDtypeStruct((NC, NP, H), jnp.float32),
    mesh=_MESH,
    scratch_types=[
        pltpu.VMEM((NCHUNK, CH), jnp.int32),
        pltpu.VMEM((GRP, CH), jnp.int32),
        pltpu.VMEM((GRP, CH), jnp.int32),
        pltpu.VMEM((GRP, CH, H), jnp.float32),
        pltpu.VMEM_SHARED((NP, H), jnp.float32),
        pltpu.SemaphoreType.DMA,
        pltpu.SemaphoreType.DMA,
        pltpu.SemaphoreType.DMA,
    ],
)

_RB = 2000


def _tc_mm_body(x_ref, w_ref, xw_ref):
    xw_ref[...] = jnp.dot(x_ref[...], w_ref[...],
                          preferred_element_type=jnp.float32)


_tc_mm = pl.pallas_call(
    _tc_mm_body,
    grid=(N // _RB,),
    in_specs=[
        pl.BlockSpec((_RB, H), lambda i: (i, 0)),
        pl.BlockSpec((H, H), lambda i: (0, 0)),
    ],
    out_specs=pl.BlockSpec((_RB, H), lambda i: (i, 0)),
    out_shape=jax.ShapeDtypeStruct((N, H), jnp.float32),
)


def _tc_scale_body(xw_ref, cnt_ref, y_ref):
    deg = cnt_ref[:, 0:1] + cnt_ref[:, 1:2] + 1.0
    y_ref[...] = xw_ref[...] * lax.rsqrt(deg)


_tc_scale = pl.pallas_call(
    _tc_scale_body,
    grid=(N // _RB,),
    in_specs=[
        pl.BlockSpec((_RB, H), lambda i: (i, 0)),
        pl.BlockSpec((_RB, NC), lambda i: (i, 0)),
    ],
    out_specs=pl.BlockSpec((_RB, H), lambda i: (i, 0)),
    out_shape=jax.ShapeDtypeStruct((N, H), jnp.float32),
)


def _tc1_body(x_ref, w_ref, cnt_ref, y_ref):
    deg = cnt_ref[:, 0:1] + cnt_ref[:, 1:2] + 1.0
    y_ref[...] = (
        jnp.dot(x_ref[...], w_ref[...], preferred_element_type=jnp.float32)
        * lax.rsqrt(deg)
    )


_tc1 = pl.pallas_call(
    _tc1_body,
    grid=(N // _RB,),
    in_specs=[
        pl.BlockSpec((_RB, H), lambda i: (i, 0)),
        pl.BlockSpec((H, H), lambda i: (0, 0)),
        pl.BlockSpec((_RB, NC), lambda i: (i, 0)),
    ],
    out_specs=pl.BlockSpec((_RB, H), lambda i: (i, 0)),
    out_shape=jax.ShapeDtypeStruct((N, H), jnp.float32),
)


def _tc2_body(a0_ref, a1_ref, y_ref, cnt_ref, x_ref, bg_ref, wl_ref, bl_ref,
              o_ref):
    deg = cnt_ref[:, 0:1] + cnt_ref[:, 1:2] + 1.0
    dinv = lax.rsqrt(deg)
    h = (a0_ref[0] + a1_ref[0] + y_ref[...]) * dinv + bg_ref[...]
    h = jnp.maximum(h, 0.0)
    o = jnp.dot(h, wl_ref[...], preferred_element_type=jnp.float32) + bl_ref[...]
    o_ref[...] = jnp.maximum(o, 0.0) + x_ref[...]


_tc2 = pl.pallas_call(
    _tc2_body,
    grid=(N // _RB,),
    in_specs=[
        pl.BlockSpec((1, _RB, H), lambda i: (0, i, 0)),
        pl.BlockSpec((1, _RB, H), lambda i: (1, i, 0)),
        pl.BlockSpec((_RB, H), lambda i: (i, 0)),
        pl.BlockSpec((_RB, NC), lambda i: (i, 0)),
        pl.BlockSpec((_RB, H), lambda i: (i, 0)),
        pl.BlockSpec((1, H), lambda i: (0, 0)),
        pl.BlockSpec((H, H), lambda i: (0, 0)),
        pl.BlockSpec((1, H), lambda i: (0, 0)),
    ],
    out_specs=pl.BlockSpec((_RB, H), lambda i: (i, 0)),
    out_shape=jax.ShapeDtypeStruct((N, H), jnp.float32),
)


def kernel(x, edge_index, W_gcn, b_gcn, W_lin, b_lin):
    src = edge_index[0].astype(jnp.int32)
    dst = edge_index[1].astype(jnp.int32)
    pad_ar = jnp.arange(EPAD, dtype=jnp.int32)
    src_p = jnp.concatenate([src, pad_ar % N])
    dst_p = jnp.concatenate([dst, N + pad_ar % (NP - N)])
    pk3 = ((dst_p << 16) | src_p).reshape(NW, NCHUNK, CH)
    dst3c = dst_p.reshape(NW, NCHUNK_C, CHC)

    cnt = _sc_count(dst3c).reshape(NC, NP)
    cntT = cnt[:, :N].T
    y = _tc1(x, W_gcn, cntT)
    agg = _sc_scatter(y, pk3)
    out = _tc2(agg, agg, y, cntT, x, b_gcn.reshape(1, H), W_lin,
               b_lin.reshape(1, H))
    return out


# speedup vs baseline: 1.0060x; 1.0060x over previous
import jax
import jax.numpy as jnp
from jax import lax
from jax.experimental import pallas as pl
from jax.experimental.pallas import tpu as pltpu
from jax.experimental.pallas import tpu_sc as plsc

N = 10000
H = 128
E = 320000

NC = 2
NS = 16
NW = NC * NS
CH = 80
NCHUNK = 126
GRP = 3
NG = NCHUNK // GRP
EPAD = NW * NCHUNK * CH - E
CHC = 112
NCHUNK_C = NCHUNK * CH // CHC
NP = 10112
WPT = NP // NS

_MESH = plsc.VectorSubcoreMesh(
    core_axis_name="c", subcore_axis_name="s", num_cores=NC, num_subcores=NS
)


def _zero16():
    return jnp.zeros((16,), jnp.float32)


def _sc_count_body(dst_hbm, out_hbm, idx_v, ones_v, zero_v, tmp_v, cnt_sh):
    c = lax.axis_index("c")
    s = lax.axis_index("s")
    wid = s * NC + c
    for j in range(CHC // 16):
        ones_v[pl.ds(j * 16, 16)] = jnp.full((16,), 1.0, jnp.float32)
    for j in range(8):
        zero_v[pl.ds(j * 16, 16)] = _zero16()
    base = s * WPT
    for k in range(4):
        pltpu.sync_copy(zero_v, cnt_sh.at[pl.ds(base + k * 128, 128)])
    pltpu.sync_copy(zero_v.at[pl.ds(0, 120)], cnt_sh.at[pl.ds(base + 512, 120)])
    plsc.subcore_barrier()
    pltpu.sync_copy(dst_hbm.at[wid], idx_v)

    def body(i, carry):
        pltpu.sync_copy(ones_v, cnt_sh.at[idx_v.at[i]], add=True)
        return carry

    lax.fori_loop(0, NCHUNK_C, body, 0)
    plsc.subcore_barrier()
    pltpu.sync_copy(cnt_sh.at[pl.ds(base, WPT)], tmp_v)
    pltpu.sync_copy(tmp_v, out_hbm.at[pl.ds(c * NP + base, WPT)])


_sc_count = pl.kernel(
    _sc_count_body,
    out_type=jax.ShapeDtypeStruct((NC * NP,), jnp.float32),
    mesh=_MESH,
    scratch_types=[
        pltpu.VMEM((NCHUNK_C, CHC), jnp.int32),
        pltpu.VMEM((CHC,), jnp.float32),
        pltpu.VMEM((128,), jnp.float32),
        pltpu.VMEM((WPT,), jnp.float32),
        pltpu.VMEM_SHARED((NP,), jnp.float32),
    ],
)


def _sc_scatter_body(y_hbm, pk_hbm, out_hbm, pk_v, sidx_v, didx_v, buf_v,
                     agg_sh, g0, g1, g2):
    gsems = (g0, g1, g2)
    c = lax.axis_index("c")
    s = lax.axis_index("s")
    wid = s * NC + c

    def zbody(r, carry):
        for j in range(8):
            buf_v[0, r, pl.ds(j * 16, 16)] = _zero16()
        return carry

    lax.fori_loop(0, CH, zbody, 0)
    base = s * WPT
    for k in range(7):
        pltpu.sync_copy(buf_v.at[0], agg_sh.at[pl.ds(base + k * CH, CH)])
    pltpu.sync_copy(buf_v.at[0, pl.ds(0, 72)],
                    agg_sh.at[pl.ds(base + 7 * CH, 72)])
    plsc.subcore_barrier()
    pltpu.sync_copy(pk_hbm.at[wid], pk_v)

    def unpack(i, b):
        for j in range(CH // 16):
            v = pk_v[i, pl.ds(j * 16, 16)]
            sidx_v[b, pl.ds(j * 16, 16)] = jnp.bitwise_and(
                v, jnp.int32(0xFFFF))
            didx_v[b, pl.ds(j * 16, 16)] = lax.shift_right_logical(v, 16)

    def fire_g(b):
        pltpu.async_copy(y_hbm.at[sidx_v.at[b]], buf_v.at[b], gsems[b])

    def wait_g(b):
        pltpu.make_async_copy(y_hbm.at[sidx_v.at[b]], buf_v.at[b],
                              gsems[b]).wait()

    for b in range(GRP):
        unpack(b, b)
        fire_g(b)

    def body(g, carry):
        i0 = g * GRP
        for b in range(GRP):
            wait_g(b)
            pltpu.sync_copy(buf_v.at[b], agg_sh.at[didx_v.at[b]], add=True)

            @pl.when(g < NG - 1)
            def _(b=b):
                unpack(i0 + GRP + b, b)
                fire_g(b)

        return carry

    lax.fori_loop(0, NG, body, 0)
    plsc.subcore_barrier()
    pltpu.sync_copy(agg_sh.at[pl.ds(base, WPT)], out_hbm.at[c, pl.ds(base, WPT)])


_sc_scatter = pl.kernel(
    _sc_scatter_body,
    out_type=jax.ShapeDtypeStruct((NC, NP, H), jnp.float32),
    mesh=_MESH,
    scratch_types=[
        pltpu.VMEM((NCHUNK, CH), jnp.int32),
        pltpu.VMEM((GRP, CH), jnp.int32),
        pltpu.VMEM((GRP, CH), jnp.int32),
        pltpu.VMEM((GRP, CH, H), jnp.float32),
        pltpu.VMEM_SHARED((NP, H), jnp.float32),
        pltpu.SemaphoreType.DMA,
        pltpu.SemaphoreType.DMA,
        pltpu.SemaphoreType.DMA,
    ],
)

_RB = 2000


def _tc_mm_body(x_ref, w_ref, xw_ref):
    xw_ref[...] = jnp.dot(x_ref[...], w_ref[...],
                          preferred_element_type=jnp.float32)


_tc_mm = pl.pallas_call(
    _tc_mm_body,
    grid=(N // _RB,),
    in_specs=[
        pl.BlockSpec((_RB, H), lambda i: (i, 0)),
        pl.BlockSpec((H, H), lambda i: (0, 0)),
    ],
    out_specs=pl.BlockSpec((_RB, H), lambda i: (i, 0)),
    out_shape=jax.ShapeDtypeStruct((N, H), jnp.float32),
)


def _tc_scale_body(xw_ref, cnt_ref, y_ref):
    deg = cnt_ref[:, 0:1] + cnt_ref[:, 1:2] + 1.0
    y_ref[...] = xw_ref[...] * lax.rsqrt(deg)


_tc_scale = pl.pallas_call(
    _tc_scale_body,
    grid=(N // _RB,),
    in_specs=[
        pl.BlockSpec((_RB, H), lambda i: (i, 0)),
        pl.BlockSpec((_RB, NC), lambda i: (i, 0)),
    ],
    out_specs=pl.BlockSpec((_RB, H), lambda i: (i, 0)),
    out_shape=jax.ShapeDtypeStruct((N, H), jnp.float32),
)


def _tc2_body(a0_ref, a1_ref, y_ref, cnt_ref, x_ref, bg_ref, wl_ref, bl_ref,
              o_ref):
    deg = cnt_ref[:, 0:1] + cnt_ref[:, 1:2] + 1.0
    dinv = lax.rsqrt(deg)
    h = (a0_ref[0] + a1_ref[0] + y_ref[...]) * dinv + bg_ref[...]
    h = jnp.maximum(h, 0.0)
    o = jnp.dot(h, wl_ref[...], preferred_element_type=jnp.float32) + bl_ref[...]
    o_ref[...] = jnp.maximum(o, 0.0) + x_ref[...]


_tc2 = pl.pallas_call(
    _tc2_body,
    grid=(N // _RB,),
    in_specs=[
        pl.BlockSpec((1, _RB, H), lambda i: (0, i, 0)),
        pl.BlockSpec((1, _RB, H), lambda i: (1, i, 0)),
        pl.BlockSpec((_RB, H), lambda i: (i, 0)),
        pl.BlockSpec((_RB, NC), lambda i: (i, 0)),
        pl.BlockSpec((_RB, H), lambda i: (i, 0)),
        pl.BlockSpec((1, H), lambda i: (0, 0)),
        pl.BlockSpec((H, H), lambda i: (0, 0)),
        pl.BlockSpec((1, H), lambda i: (0, 0)),
    ],
    out_specs=pl.BlockSpec((_RB, H), lambda i: (i, 0)),
    out_shape=jax.ShapeDtypeStruct((N, H), jnp.float32),
)


def kernel(x, edge_index, W_gcn, b_gcn, W_lin, b_lin):
    src = edge_index[0].astype(jnp.int32)
    dst = edge_index[1].astype(jnp.int32)
    pad_ar = jnp.arange(EPAD, dtype=jnp.int32)
    src_p = jnp.concatenate([src, pad_ar % N])
    dst_p = jnp.concatenate([dst, N + pad_ar % (NP - N)])
    pk3 = ((dst_p << 16) | src_p).reshape(NW, NCHUNK, CH)
    dst3c = dst_p.reshape(NW, NCHUNK_C, CHC)

    xw = _tc_mm(x, W_gcn)
    cnt = _sc_count(dst3c).reshape(NC, NP)
    cntT = cnt[:, :N].T
    y = _tc_scale(xw, cntT)
    agg = _sc_scatter(y, pk3)
    out = _tc2(agg, agg, y, cntT, x, b_gcn.reshape(1, H), W_lin,
               b_lin.reshape(1, H))
    return out


# speedup vs baseline: 1.0136x; 1.0076x over previous
import jax
import jax.numpy as jnp
from jax import lax
from jax.experimental import pallas as pl
from jax.experimental.pallas import tpu as pltpu
from jax.experimental.pallas import tpu_sc as plsc

N = 10000
H = 128
E = 320000

NC = 2
NS = 16
NW = NC * NS
CH = 80
NCHUNK = 126
GRP = 3
NG = NCHUNK // GRP
EPAD = NW * NCHUNK * CH - E
CHC = 112
NCHUNK_C = NCHUNK * CH // CHC
NP = 10112
WPT = NP // NS

_MESH = plsc.VectorSubcoreMesh(
    core_axis_name="c", subcore_axis_name="s", num_cores=NC, num_subcores=NS
)


def _zero16():
    return jnp.zeros((16,), jnp.float32)


def _sc_count_body(dst_hbm, out_hbm, idx_v, ones_v, zero_v, tmp_v, cnt_sh):
    c = lax.axis_index("c")
    s = lax.axis_index("s")
    wid = s * NC + c
    for j in range(CHC // 16):
        ones_v[pl.ds(j * 16, 16)] = jnp.full((16,), 1.0, jnp.float32)
    for j in range(8):
        zero_v[pl.ds(j * 16, 16)] = _zero16()
    base = s * WPT
    for k in range(4):
        pltpu.sync_copy(zero_v, cnt_sh.at[pl.ds(base + k * 128, 128)])
    pltpu.sync_copy(zero_v.at[pl.ds(0, 120)], cnt_sh.at[pl.ds(base + 512, 120)])
    plsc.subcore_barrier()
    pltpu.sync_copy(dst_hbm.at[wid], idx_v)

    def body(i, carry):
        pltpu.sync_copy(ones_v, cnt_sh.at[idx_v.at[i]], add=True)
        return carry

    lax.fori_loop(0, NCHUNK_C, body, 0)
    plsc.subcore_barrier()
    pltpu.sync_copy(cnt_sh.at[pl.ds(base, WPT)], tmp_v)
    pltpu.sync_copy(tmp_v, out_hbm.at[pl.ds(c * NP + base, WPT)])


_sc_count = pl.kernel(
    _sc_count_body,
    out_type=jax.ShapeDtypeStruct((NC * NP,), jnp.float32),
    mesh=_MESH,
    scratch_types=[
        pltpu.VMEM((NCHUNK_C, CHC), jnp.int32),
        pltpu.VMEM((CHC,), jnp.float32),
        pltpu.VMEM((128,), jnp.float32),
        pltpu.VMEM((WPT,), jnp.float32),
        pltpu.VMEM_SHARED((NP,), jnp.float32),
    ],
)


def _sc_scatter_body(y_hbm, pk_hbm, out_hbm, pk_v, sidx_v, didx_v, buf_v,
                     agg_sh, g0, g1, g2):
    gsems = (g0, g1, g2)
    c = lax.axis_index("c")
    s = lax.axis_index("s")
    wid = s * NC + c

    def zbody(r, carry):
        for j in range(8):
            buf_v[0, r, pl.ds(j * 16, 16)] = _zero16()
        return carry

    lax.fori_loop(0, CH, zbody, 0)
    base = s * WPT
    for k in range(7):
        pltpu.sync_copy(buf_v.at[0], agg_sh.at[pl.ds(base + k * CH, CH)])
    pltpu.sync_copy(buf_v.at[0, pl.ds(0, 72)],
                    agg_sh.at[pl.ds(base + 7 * CH, 72)])
    plsc.subcore_barrier()
    pltpu.sync_copy(pk_hbm.at[wid], pk_v)

    def unpack(i, b):
        for j in range(CH // 16):
            v = pk_v[i, pl.ds(j * 16, 16)]
            sidx_v[b, pl.ds(j * 16, 16)] = jnp.bitwise_and(
                v, jnp.int32(0xFFFF))
            didx_v[b, pl.ds(j * 16, 16)] = lax.shift_right_logical(v, 16)

    def fire_g(b):
        pltpu.async_copy(y_hbm.at[sidx_v.at[b]], buf_v.at[b], gsems[b])

    def wait_g(b):
        pltpu.make_async_copy(y_hbm.at[sidx_v.at[b]], buf_v.at[b],
                              gsems[b]).wait()

    for b in range(GRP):
        unpack(b, b)
        fire_g(b)

    def body(g, carry):
        i0 = g * GRP
        for b in range(GRP):
            wait_g(b)
            pltpu.sync_copy(buf_v.at[b], agg_sh.at[didx_v.at[b]], add=True)

            @pl.when(g < NG - 1)
            def _(b=b):
                unpack(i0 + GRP + b, b)
                fire_g(b)

        return carry

    lax.fori_loop(0, NG, body, 0)
    plsc.subcore_barrier()
    pltpu.sync_copy(agg_sh.at[pl.ds(base, WPT)], out_hbm.at[c, pl.ds(base, WPT)])


_sc_scatter = pl.kernel(
    _sc_scatter_body,
    out_type=jax.ShapeDtypeStruct((NC, NP, H), jnp.float32),
    mesh=_MESH,
    scratch_types=[
        pltpu.VMEM((NCHUNK, CH), jnp.int32),
        pltpu.VMEM((GRP, CH), jnp.int32),
        pltpu.VMEM((GRP, CH), jnp.int32),
        pltpu.VMEM((GRP, CH, H), jnp.float32),
        pltpu.VMEM_SHARED((NP, H), jnp.float32),
        pltpu.SemaphoreType.DMA,
        pltpu.SemaphoreType.DMA,
        pltpu.SemaphoreType.DMA,
    ],
)

_RB = 2000


def _tc_mm_body(x_ref, w_ref, xw_ref):
    xw_ref[...] = jnp.dot(x_ref[...], w_ref[...],
                          preferred_element_type=jnp.float32)


_tc_mm = pl.pallas_call(
    _tc_mm_body,
    grid=(N // _RB,),
    in_specs=[
        pl.BlockSpec((_RB, H), lambda i: (i, 0)),
        pl.BlockSpec((H, H), lambda i: (0, 0)),
    ],
    out_specs=pl.BlockSpec((_RB, H), lambda i: (i, 0)),
    out_shape=jax.ShapeDtypeStruct((N, H), jnp.float32),
)


def _tc_scale_body(xw_ref, cnt_ref, y_ref):
    dinv = lax.rsqrt(cnt_ref[...] + 1.0)
    y_ref[...] = xw_ref[...] * dinv


_tc_scale = pl.pallas_call(
    _tc_scale_body,
    grid=(N // _RB,),
    in_specs=[
        pl.BlockSpec((_RB, H), lambda i: (i, 0)),
        pl.BlockSpec((_RB, 1), lambda i: (i, 0)),
    ],
    out_specs=pl.BlockSpec((_RB, H), lambda i: (i, 0)),
    out_shape=jax.ShapeDtypeStruct((N, H), jnp.float32),
)


def _tc2_body(a0_ref, a1_ref, y_ref, cnt_ref, x_ref, bg_ref, wl_ref, bl_ref,
              o_ref):
    dinv = lax.rsqrt(cnt_ref[...] + 1.0)
    h = (a0_ref[0] + a1_ref[0] + y_ref[...]) * dinv + bg_ref[...]
    h = jnp.maximum(h, 0.0)
    o = jnp.dot(h, wl_ref[...], preferred_element_type=jnp.float32) + bl_ref[...]
    o_ref[...] = jnp.maximum(o, 0.0) + x_ref[...]


_tc2 = pl.pallas_call(
    _tc2_body,
    grid=(N // _RB,),
    in_specs=[
        pl.BlockSpec((1, _RB, H), lambda i: (0, i, 0)),
        pl.BlockSpec((1, _RB, H), lambda i: (1, i, 0)),
        pl.BlockSpec((_RB, H), lambda i: (i, 0)),
        pl.BlockSpec((_RB, 1), lambda i: (i, 0)),
        pl.BlockSpec((_RB, H), lambda i: (i, 0)),
        pl.BlockSpec((1, H), lambda i: (0, 0)),
        pl.BlockSpec((H, H), lambda i: (0, 0)),
        pl.BlockSpec((1, H), lambda i: (0, 0)),
    ],
    out_specs=pl.BlockSpec((_RB, H), lambda i: (i, 0)),
    out_shape=jax.ShapeDtypeStruct((N, H), jnp.float32),
)


def kernel(x, edge_index, W_gcn, b_gcn, W_lin, b_lin):
    src = edge_index[0].astype(jnp.int32)
    dst = edge_index[1].astype(jnp.int32)
    pad_ar = jnp.arange(EPAD, dtype=jnp.int32)
    src_p = jnp.concatenate([src, pad_ar % N])
    dst_p = jnp.concatenate([dst, N + pad_ar % (NP - N)])
    pk3 = ((dst_p << 16) | src_p).reshape(NW, NCHUNK, CH)
    dst3c = dst_p.reshape(NW, NCHUNK_C, CHC)

    xw = _tc_mm(x, W_gcn)
    cnt = _sc_count(dst3c).reshape(NC, NP)
    degs = (cnt[0, :N] + cnt[1, :N]).reshape(N, 1)
    y = _tc_scale(xw, degs)
    agg = _sc_scatter(y, pk3)
    out = _tc2(agg, agg, y, degs, x, b_gcn.reshape(1, H), W_lin,
               b_lin.reshape(1, H))
    return out
